# baseline (speedup 1.0000x reference)
import functools

import jax
import jax.numpy as jnp
from jax import lax
from jax.experimental import pallas as pl
from jax.experimental.pallas import tpu as pltpu
from jax.experimental.pallas import tpu_sc as plsc

NUM_EMB = 100000
DIM = 128
WPR = DIM // 4

NUM_CORES = 2
NUM_SUBCORES = 16
NW = NUM_CORES * NUM_SUBCORES

CHUNK = 128


def _sc_gather(table_i32, idx_flat, scaler):
    B = idx_flat.shape[0]
    assert B % NW == 0
    b_per_w = B // NW
    assert b_per_w % CHUNK == 0
    n_chunks = b_per_w // CHUNK

    mesh = plsc.VectorSubcoreMesh(
        core_axis_name="c",
        subcore_axis_name="s",
        num_cores=NUM_CORES,
        num_subcores=NUM_SUBCORES,
    )

    @functools.partial(
        pl.kernel,
        out_type=[
            jax.ShapeDtypeStruct((B, WPR), jnp.int32),
            jax.ShapeDtypeStruct((B,), jnp.float32),
        ],
        mesh=mesh,
        compiler_params=pltpu.CompilerParams(use_tc_tiling_on_sc=False),
        scratch_types=[
            pltpu.VMEM((n_chunks, CHUNK), jnp.int32),
            pltpu.VMEM((CHUNK, WPR), jnp.int32),
            pltpu.VMEM((CHUNK,), jnp.float32),
            pltpu.SemaphoreType.DMA,
            pltpu.SemaphoreType.DMA,
        ],
    )
    def k(tab_hbm, idx_hbm, sca_hbm, rows_out, sca_out,
          idx_v, rows_v, sc_v, sem_r, sem_s):
        wid = lax.axis_index("s") * NUM_CORES + lax.axis_index("c")
        base = wid * b_per_w
        pltpu.sync_copy(idx_hbm.at[wid], idx_v)

        @pl.loop(0, n_chunks)
        def _chunk(j):
            off = base + j * CHUNK
            cp_r = pltpu.async_copy(tab_hbm.at[idx_v.at[j]], rows_v, sem_r)
            cp_s = pltpu.async_copy(sca_hbm.at[idx_v.at[j]], sc_v, sem_s)
            cp_r.wait()
            cp_s.wait()
            pltpu.sync_copy(rows_v, rows_out.at[pl.ds(off, CHUNK)])
            pltpu.sync_copy(sc_v, sca_out.at[pl.ds(off, CHUNK)])

    idx3 = idx_flat.reshape(NW, n_chunks, CHUNK)
    return k(table_i32, idx3, scaler)


def _tc_dequant(rows_i8, scal):
    B = rows_i8.shape[0]
    BLK = 2048
    assert B % BLK == 0

    def body(rows_ref, sc_ref, out_ref):
        out_ref[...] = rows_ref[...].astype(jnp.float32) * sc_ref[...]

    return pl.pallas_call(
        body,
        grid=(B // BLK,),
        in_specs=[
            pl.BlockSpec((BLK, DIM), lambda i: (i, 0)),
            pl.BlockSpec((BLK, 1), lambda i: (i, 0)),
        ],
        out_specs=pl.BlockSpec((BLK, DIM), lambda i: (i, 0)),
        out_shape=jax.ShapeDtypeStruct((B, DIM), jnp.float32),
    )(rows_i8, scal)


def kernel(x, weight, weight_scaler):
    B0, S = x.shape
    idx_flat = x.reshape(-1).astype(jnp.int32)
    table_i32 = lax.bitcast_convert_type(
        weight.reshape(NUM_EMB, WPR, 4), jnp.int32
    )
    rows_i32, scal = _sc_gather(table_i32, idx_flat, weight_scaler)
    rows_i8 = lax.bitcast_convert_type(rows_i32, jnp.int8).reshape(-1, DIM)
    out = _tc_dequant(rows_i8, scal.reshape(-1, 1))
    return out.reshape(B0, S, DIM)


# speedup vs baseline: 2.1088x; 2.1088x over previous
import functools

import jax
import jax.numpy as jnp
from jax import lax
from jax.experimental import pallas as pl
from jax.experimental.pallas import tpu as pltpu
from jax.experimental.pallas import tpu_sc as plsc

NUM_EMB = 100000
DIM = 128

NUM_CORES = 2
NUM_SUBCORES = 16
NW = NUM_CORES * NUM_SUBCORES

CHUNK = 128


def _sc_gather(table_i8, idx3, scaler):
    _, n_chunks, _ = idx3.shape
    B = NW * n_chunks * CHUNK
    b_per_w = n_chunks * CHUNK

    mesh = plsc.VectorSubcoreMesh(
        core_axis_name="c",
        subcore_axis_name="s",
        num_cores=NUM_CORES,
        num_subcores=NUM_SUBCORES,
    )

    @functools.partial(
        pl.kernel,
        out_type=[
            jax.ShapeDtypeStruct((B, DIM), jnp.int8),
            jax.ShapeDtypeStruct((B,), jnp.float32),
        ],
        mesh=mesh,
        compiler_params=pltpu.CompilerParams(use_tc_tiling_on_sc=False),
        scratch_types=[
            pltpu.VMEM((n_chunks, CHUNK), jnp.int32),
            pltpu.VMEM((2, CHUNK, DIM), jnp.int8),
            pltpu.VMEM((2, CHUNK), jnp.float32),
            pltpu.SemaphoreType.DMA,
            pltpu.SemaphoreType.DMA,
            pltpu.SemaphoreType.DMA,
            pltpu.SemaphoreType.DMA,
        ],
    )
    def k(tab_hbm, idx_hbm, sca_hbm, rows_out, sca_out,
          idx_v, rows_v, sc_v, sem_r0, sem_r1, sem_s0, sem_s1):
        wid = lax.axis_index("s") * NUM_CORES + lax.axis_index("c")
        base = wid * b_per_w
        sem_r = (sem_r0, sem_r1)
        sem_s = (sem_s0, sem_s1)
        pltpu.sync_copy(idx_hbm.at[wid], idx_v)

        def start(j, slot):
            pltpu.async_copy(
                tab_hbm.at[idx_v.at[j]], rows_v.at[slot], sem_r[slot])
            pltpu.async_copy(
                sca_hbm.at[idx_v.at[j]], sc_v.at[slot], sem_s[slot])

        def finish(j, slot):
            pltpu.make_async_copy(
                tab_hbm.at[idx_v.at[j]], rows_v.at[slot], sem_r[slot]).wait()
            pltpu.make_async_copy(
                sca_hbm.at[idx_v.at[j]], sc_v.at[slot], sem_s[slot]).wait()
            off = base + j * CHUNK
            pltpu.sync_copy(rows_v.at[slot], rows_out.at[pl.ds(off, CHUNK)])
            pltpu.sync_copy(sc_v.at[slot], sca_out.at[pl.ds(off, CHUNK)])

        start(0, 0)
        start(1, 1)

        @pl.loop(0, n_chunks - 2, step=2)
        def _chunk(j):
            for b in range(2):
                finish(j + b, b)
                start(j + 2 + b, b)

        for b in range(2):
            finish(n_chunks - 2 + b, b)

    return k(table_i8, idx3, scaler)


def _tc_dequant(rows_i8, scal):
    B = rows_i8.shape[0]
    BLK = 2048
    assert B % BLK == 0

    def body(rows_ref, sc_ref, out_ref):
        out_ref[...] = rows_ref[...].astype(jnp.float32) * sc_ref[...]

    return pl.pallas_call(
        body,
        grid=(B // BLK,),
        in_specs=[
            pl.BlockSpec((BLK, DIM), lambda i: (i, 0)),
            pl.BlockSpec((BLK, 1), lambda i: (i, 0)),
        ],
        out_specs=pl.BlockSpec((BLK, DIM), lambda i: (i, 0)),
        out_shape=jax.ShapeDtypeStruct((B, DIM), jnp.float32),
    )(rows_i8, scal)


def kernel(x, weight, weight_scaler):
    B0, S = x.shape
    B = B0 * S
    b_per_w = B // NW
    idx3 = x.astype(jnp.int32).reshape(NW, b_per_w // CHUNK, CHUNK)
    rows_i8, scal = _sc_gather(weight, idx3, weight_scaler)
    out = _tc_dequant(rows_i8, scal.reshape(-1, 1))
    return out.reshape(B0, S, DIM)


# speedup vs baseline: 5.7198x; 2.7124x over previous
import functools

import jax
import jax.numpy as jnp
from jax import lax
from jax.experimental import pallas as pl
from jax.experimental.pallas import tpu as pltpu
from jax.experimental.pallas import tpu_sc as plsc

NUM_EMB = 100000
DIM = 128

NUM_CORES = 2
NUM_SUBCORES = 16
NW = NUM_CORES * NUM_SUBCORES

CHUNK = 128


def _tc_dequant_table(weight, scaler):
    BLKT = 4000
    assert NUM_EMB % BLKT == 0

    def body(w_ref, s_ref, o_ref):
        o_ref[...] = w_ref[...].astype(jnp.float32) * s_ref[...]

    return pl.pallas_call(
        body,
        grid=(NUM_EMB // BLKT,),
        in_specs=[
            pl.BlockSpec((BLKT, DIM), lambda i: (i, 0)),
            pl.BlockSpec((BLKT, 1), lambda i: (i, 0)),
        ],
        out_specs=pl.BlockSpec((BLKT, DIM), lambda i: (i, 0)),
        out_shape=jax.ShapeDtypeStruct((NUM_EMB, DIM), jnp.float32),
    )(weight, scaler.reshape(NUM_EMB, 1))


def _sc_gather(table_f32, idx3):
    _, n_chunks, _ = idx3.shape
    B = NW * n_chunks * CHUNK
    b_per_w = n_chunks * CHUNK

    mesh = plsc.VectorSubcoreMesh(
        core_axis_name="c",
        subcore_axis_name="s",
        num_cores=NUM_CORES,
        num_subcores=NUM_SUBCORES,
    )

    @functools.partial(
        pl.kernel,
        out_type=jax.ShapeDtypeStruct((B, DIM), jnp.float32),
        mesh=mesh,
        compiler_params=pltpu.CompilerParams(use_tc_tiling_on_sc=False),
        scratch_types=[
            pltpu.VMEM((n_chunks, CHUNK), jnp.int32),
            pltpu.VMEM((2, CHUNK, DIM), jnp.float32),
            pltpu.SemaphoreType.DMA,
            pltpu.SemaphoreType.DMA,
        ],
    )
    def k(tab_hbm, idx_hbm, rows_out, idx_v, rows_v, sem0, sem1):
        wid = lax.axis_index("s") * NUM_CORES + lax.axis_index("c")
        base = wid * b_per_w
        sem = (sem0, sem1)
        pltpu.sync_copy(idx_hbm.at[wid], idx_v)

        def start(j, slot):
            pltpu.async_copy(
                tab_hbm.at[idx_v.at[j]], rows_v.at[slot], sem[slot])

        def finish(j, slot):
            pltpu.make_async_copy(
                tab_hbm.at[idx_v.at[j]], rows_v.at[slot], sem[slot]).wait()
            off = base + j * CHUNK
            pltpu.sync_copy(rows_v.at[slot], rows_out.at[pl.ds(off, CHUNK)])

        start(0, 0)
        start(1, 1)

        @pl.loop(0, n_chunks - 2, step=2)
        def _chunk(j):
            for b in range(2):
                finish(j + b, b)
                start(j + 2 + b, b)

        for b in range(2):
            finish(n_chunks - 2 + b, b)

    return k(table_f32, idx3)


def kernel(x, weight, weight_scaler):
    B0, S = x.shape
    B = B0 * S
    b_per_w = B // NW
    idx3 = x.astype(jnp.int32).reshape(NW, b_per_w // CHUNK, CHUNK)
    table_f32 = _tc_dequant_table(weight, weight_scaler)
    out = _sc_gather(table_f32, idx3)
    return out.reshape(B0, S, DIM)


# speedup vs baseline: 7.9304x; 1.3865x over previous
import functools

import jax
import jax.numpy as jnp
from jax import lax
from jax.experimental import pallas as pl
from jax.experimental.pallas import tpu as pltpu
from jax.experimental.pallas import tpu_sc as plsc

NUM_EMB = 100000
DIM = 128

NUM_CORES = 2
NUM_SUBCORES = 16
NW = NUM_CORES * NUM_SUBCORES

CHUNK = 128


def _tc_dequant_table(weight, scaler):
    BLKT = 4096

    def body(w_ref, s_ref, o_ref):
        s = s_ref[...].reshape(BLKT, 1)
        o_ref[...] = w_ref[...].astype(jnp.float32) * s

    return pl.pallas_call(
        body,
        grid=(pl.cdiv(NUM_EMB, BLKT),),
        in_specs=[
            pl.BlockSpec((BLKT, DIM), lambda i: (i, 0)),
            pl.BlockSpec((BLKT,), lambda i: (i,)),
        ],
        out_specs=pl.BlockSpec((BLKT, DIM), lambda i: (i, 0)),
        out_shape=jax.ShapeDtypeStruct((NUM_EMB, DIM), jnp.float32),
    )(weight, scaler)


def _sc_gather(table_f32, idx3):
    _, n_chunks, _ = idx3.shape
    B = NW * n_chunks * CHUNK
    b_per_w = n_chunks * CHUNK

    mesh = plsc.VectorSubcoreMesh(
        core_axis_name="c",
        subcore_axis_name="s",
        num_cores=NUM_CORES,
        num_subcores=NUM_SUBCORES,
    )

    @functools.partial(
        pl.kernel,
        out_type=jax.ShapeDtypeStruct((B, DIM), jnp.float32),
        mesh=mesh,
        compiler_params=pltpu.CompilerParams(use_tc_tiling_on_sc=False),
        scratch_types=[
            pltpu.VMEM((n_chunks, CHUNK), jnp.int32),
            pltpu.VMEM((2, CHUNK, DIM), jnp.float32),
            pltpu.SemaphoreType.DMA,
            pltpu.SemaphoreType.DMA,
        ],
    )
    def k(tab_hbm, idx_hbm, rows_out, idx_v, rows_v, sem0, sem1):
        wid = lax.axis_index("s") * NUM_CORES + lax.axis_index("c")
        base = wid * b_per_w
        sem = (sem0, sem1)
        pltpu.sync_copy(idx_hbm.at[wid], idx_v)

        def start(j, slot):
            pltpu.async_copy(
                tab_hbm.at[idx_v.at[j]], rows_v.at[slot], sem[slot])

        def finish(j, slot):
            pltpu.make_async_copy(
                tab_hbm.at[idx_v.at[j]], rows_v.at[slot], sem[slot]).wait()
            off = base + j * CHUNK
            pltpu.sync_copy(rows_v.at[slot], rows_out.at[pl.ds(off, CHUNK)])

        start(0, 0)
        start(1, 1)

        @pl.loop(0, n_chunks - 2, step=2)
        def _chunk(j):
            for b in range(2):
                finish(j + b, b)
                start(j + 2 + b, b)

        for b in range(2):
            finish(n_chunks - 2 + b, b)

    return k(table_f32, idx3)


def kernel(x, weight, weight_scaler):
    B0, S = x.shape
    B = B0 * S
    b_per_w = B // NW
    idx3 = x.astype(jnp.int32).reshape(NW, b_per_w // CHUNK, CHUNK)
    table_f32 = _tc_dequant_table(weight, weight_scaler)
    out = _sc_gather(table_f32, idx3)
    return out.reshape(B0, S, DIM)
